# baseline (speedup 1.0000x reference)
import functools

import jax
import jax.numpy as jnp
from jax import lax
from jax.experimental import pallas as pl
from jax.experimental.pallas import tpu as pltpu
from jax.experimental.pallas import tpu_sc as plsc

B = 16384
D = 64
NC = 2
NS = 16
NW = NC * NS
BPW = B // NW
GCHUNK = 128
NG = BPW // GCHUNK
LANES = 16

_mesh = plsc.VectorSubcoreMesh(core_axis_name="c", subcore_axis_name="s")


@functools.partial(
    pl.kernel,
    mesh=_mesh,
    out_type=jax.ShapeDtypeStruct((B, D), jnp.float32),
    scratch_types=[
        pltpu.VMEM((BPW,), jnp.int32),
        pltpu.VMEM((BPW, D), jnp.float32),
        pltpu.SemaphoreType.DMA,
    ],
    compiler_params=pltpu.CompilerParams(use_tc_tiling_on_sc=False),
)
def _gather_tanh(idx_hbm, table_hbm, out_hbm, idx_v, rows_v, sem):
    wid = lax.axis_index("s") * NC + lax.axis_index("c")
    base = wid * BPW

    pltpu.sync_copy(idx_hbm.at[pl.ds(base, BPW)], idx_v)

    copies = []
    for g in range(NG):
        copies.append(
            pltpu.async_copy(
                table_hbm.at[idx_v.at[pl.ds(g * GCHUNK, GCHUNK)]],
                rows_v.at[pl.ds(g * GCHUNK, GCHUNK), :],
                sem,
            )
        )
    for c in copies:
        c.wait()

    def body(i, _):
        for j in range(D // LANES):
            x = rows_v[i, pl.ds(j * LANES, LANES)]
            e = jnp.exp(jnp.abs(x) * -2.0)
            rows_v[i, pl.ds(j * LANES, LANES)] = (
                jnp.sign(x) * ((1.0 - e) / (1.0 + e))
            )
        return 0

    lax.fori_loop(0, BPW, body, 0)

    pltpu.sync_copy(rows_v, out_hbm.at[pl.ds(base, BPW)])


def kernel(topics, table):
    out = _gather_tanh(topics.astype(jnp.int32), table)
    return out.reshape(B, 1, D)
